# baseline (speedup 1.0000x reference)
import functools

import jax
import jax.numpy as jnp
from jax import lax
from jax.experimental import pallas as pl
from jax.experimental.pallas import tpu as pltpu
from jax.experimental.pallas import tpu_sc as plsc

_NODE = 100000
_DIM = 64
_B = 16384
_NEG = 5

_NC = 2
_NS = 16
_NW = _NC * _NS
_CHUNK = _B // _NW
_SUB = 64
_NSUB = _CHUNK // _SUB
_PAIRW = 2 * _DIM

_UNROLL = 4


def _sc_body(vi_hbm, vj_hbm, ng_hbm, emb_hbm, ctx_hbm, out_hbm,
             idx_i, idx_j, idx_n, hidx_i, hidx_j, hidx_n,
             vi_a, vj_a, ng_a, vi_b, vj_b, ng_b, dots, sem_a, sem_b):
    cid = lax.axis_index("c")
    sid = lax.axis_index("s")
    wid = sid * _NC + cid

    pltpu.sync_copy(vi_hbm.at[pl.ds(wid * _CHUNK, _CHUNK)], idx_i)
    pltpu.sync_copy(vj_hbm.at[pl.ds(wid * _CHUNK, _CHUNK)], idx_j)
    pltpu.sync_copy(ng_hbm.at[pl.ds(wid * _CHUNK * _NEG, _CHUNK * _NEG)], idx_n)

    iota = lax.iota(jnp.int32, 16)

    def halve_ij(t, carry):
        sl = pl.ds(t * 16, 16)
        hidx_i[sl] = lax.shift_right_logical(idx_i[sl], 1)
        hidx_j[sl] = lax.shift_right_logical(idx_j[sl], 1)
        return carry

    lax.fori_loop(0, _CHUNK // 16, halve_ij, 0)

    def halve_n(t, carry):
        sl = pl.ds(t * 16, 16)
        hidx_n[sl] = lax.shift_right_logical(idx_n[sl], 1)
        return carry

    lax.fori_loop(0, _CHUNK * _NEG // 16, halve_n, 0)

    bufs = [(vi_a, vj_a, ng_a, sem_a), (vi_b, vj_b, ng_b, sem_b)]

    def fire(s):
        vi_r, vj_r, ng_r, sem = bufs[s % 2]
        cps = [pltpu.async_copy(emb_hbm.at[hidx_i.at[pl.ds(s * _SUB, _SUB)]],
                                vi_r, sem),
               pltpu.async_copy(ctx_hbm.at[hidx_j.at[pl.ds(s * _SUB, _SUB)]],
                                vj_r, sem)]
        cps += [pltpu.async_copy(
                    ctx_hbm.at[hidx_n.at[pl.ds((s * _NEG + k) * _SUB, _SUB)]],
                    ng_r.at[pl.ds(k * _SUB, _SUB)], sem)
                for k in range(_NEG)]
        return cps

    inflight = fire(0)
    for s in range(_NSUB):
        for c in inflight:
            c.wait()
        inflight = fire(s + 1) if s + 1 < _NSUB else []
        vi_r, vj_r, ng_r, _ = bufs[s % 2]

        def group(g, carry2):
            rows = g * 16 + iota
            base = s * _SUB + g * 16
            par_i = (idx_i[pl.ds(base, 16)] & 1) * _DIM
            par_j = (idx_j[pl.ds(base, 16)] & 1) * _DIM
            ngrows = []
            par_n = []
            for k in range(_NEG):
                pos = (base + iota) * _NEG + k
                par_n.append((plsc.load_gather(idx_n, [pos]) & 1) * _DIM)
                ngrows.append(rows * _NEG + k)

            def dstep(t, accs):
                d0 = t * _UNROLL
                for j in range(_UNROLL):
                    dsk = (jnp.full((16,), d0 + j, jnp.int32) + iota) & (_DIM - 1)
                    vi = plsc.load_gather(vi_r, [rows, par_i + dsk])
                    vj = plsc.load_gather(vj_r, [rows, par_j + dsk])
                    out = [accs[0] + vi * vj]
                    for k in range(_NEG):
                        ng = plsc.load_gather(ng_r, [ngrows[k], par_n[k] + dsk])
                        out.append(accs[k + 1] + vi * ng)
                    accs = tuple(out)
                return accs

            zero = jnp.zeros((16,), jnp.float32)
            accs = lax.fori_loop(0, _DIM // _UNROLL, dstep, (zero,) * 6)
            for t in range(6):
                dots[t, pl.ds(base, 16)] = accs[t]
            return carry2

        lax.fori_loop(0, _SUB // 16, group, 0)

    for t in range(6):
        pltpu.sync_copy(dots.at[t], out_hbm.at[t, pl.ds(wid * _CHUNK, _CHUNK)])


_sc_dots = functools.partial(
    pl.kernel,
    out_type=jax.ShapeDtypeStruct((6, _B), jnp.float32),
    mesh=plsc.VectorSubcoreMesh(core_axis_name="c", subcore_axis_name="s"),
    scratch_types=[
        pltpu.VMEM((_CHUNK,), jnp.int32),
        pltpu.VMEM((_CHUNK,), jnp.int32),
        pltpu.VMEM((_CHUNK * _NEG,), jnp.int32),
        pltpu.VMEM((_CHUNK,), jnp.int32),
        pltpu.VMEM((_CHUNK,), jnp.int32),
        pltpu.VMEM((_CHUNK * _NEG,), jnp.int32),
        pltpu.VMEM((_SUB, _PAIRW), jnp.float32),
        pltpu.VMEM((_SUB, _PAIRW), jnp.float32),
        pltpu.VMEM((_SUB * _NEG, _PAIRW), jnp.float32),
        pltpu.VMEM((_SUB, _PAIRW), jnp.float32),
        pltpu.VMEM((_SUB, _PAIRW), jnp.float32),
        pltpu.VMEM((_SUB * _NEG, _PAIRW), jnp.float32),
        pltpu.VMEM((6, _CHUNK), jnp.float32),
        pltpu.SemaphoreType.DMA,
        pltpu.SemaphoreType.DMA,
    ],
    compiler_params=pltpu.CompilerParams(
        needs_layout_passes=False, use_tc_tiling_on_sc=True),
)(_sc_body)


def _finish_body(dots_ref, out_ref):
    x = dots_ref[...]
    pos = x[0:1, :]
    neg = x[1:6, :]

    def logsig(v):
        return jnp.minimum(v, 0.0) - jnp.log1p(jnp.exp(-jnp.abs(v)))

    tot = jnp.sum(logsig(pos)) + jnp.sum(logsig(-neg))
    out_ref[0, 0] = -tot / _B


_finish = pl.pallas_call(
    _finish_body,
    out_shape=jax.ShapeDtypeStruct((1, 1), jnp.float32),
    out_specs=pl.BlockSpec(memory_space=pltpu.SMEM),
)


def kernel(emb_table, ctx_table, v_i, v_j, negative):
    vi_r = v_i.astype(jnp.int32)
    vj_r = v_j.astype(jnp.int32)
    ng_r = negative.astype(jnp.int32).reshape(-1)
    emb_p = emb_table.reshape(_NODE // 2, _PAIRW)
    ctx_p = ctx_table.reshape(_NODE // 2, _PAIRW)
    dots = _sc_dots(vi_r, vj_r, ng_r, emb_p, ctx_p)
    return _finish(dots)[0, 0]


# speedup vs baseline: 1.1032x; 1.1032x over previous
import functools

import jax
import jax.numpy as jnp
from jax import lax
from jax.experimental import pallas as pl
from jax.experimental.pallas import tpu as pltpu
from jax.experimental.pallas import tpu_sc as plsc

_NODE = 100000
_DIM = 64
_B = 16384
_NEG = 5

_NC = 2
_NS = 16
_NW = _NC * _NS
_CHUNK = _B // _NW
_SUB = 128
_NSUB = _CHUNK // _SUB


_UNROLL = 4


def _sc_body(vi_hbm, vj_hbm, ng_hbm, emb_hbm, ctx_hbm, out_hbm,
             idx_i, idx_j, idx_n,
             vi_a, vj_a, ng_a, vi_b, vj_b, ng_b, dots, sem_a, sem_b):
    cid = lax.axis_index("c")
    sid = lax.axis_index("s")
    wid = sid * _NC + cid

    pltpu.sync_copy(vi_hbm.at[pl.ds(wid * _CHUNK, _CHUNK)], idx_i)
    pltpu.sync_copy(vj_hbm.at[pl.ds(wid * _CHUNK, _CHUNK)], idx_j)
    for k in range(_NEG):
        pltpu.sync_copy(ng_hbm.at[k, pl.ds(wid * _CHUNK, _CHUNK)], idx_n.at[k])

    iota = lax.iota(jnp.int32, 16)
    bufs = [(vi_a, vj_a, ng_a, sem_a), (vi_b, vj_b, ng_b, sem_b)]

    def fire(s):
        vi_r, vj_r, ng_r, sem = bufs[s % 2]
        cps = [pltpu.async_copy(emb_hbm.at[idx_i.at[pl.ds(s * _SUB, _SUB)]],
                                vi_r, sem),
               pltpu.async_copy(ctx_hbm.at[idx_j.at[pl.ds(s * _SUB, _SUB)]],
                                vj_r, sem)]
        cps += [pltpu.async_copy(
                    ctx_hbm.at[idx_n.at[k, pl.ds(s * _SUB, _SUB)]],
                    ng_r.at[pl.ds(k * _SUB, _SUB)], sem)
                for k in range(_NEG)]
        return cps

    inflight = fire(0)
    for s in range(_NSUB):
        for c in inflight:
            c.wait()
        inflight = fire(s + 1) if s + 1 < _NSUB else []
        vi_r, vj_r, ng_r, _ = bufs[s % 2]

        def group(g, carry2):
            rows = g * 16 + iota
            ngrows = [rows + k * _SUB for k in range(_NEG)]

            def dstep(t, accs):
                d0 = t * _UNROLL
                for j in range(_UNROLL):
                    dcol = (jnp.full((16,), d0 + j, jnp.int32) + iota) & (_DIM - 1)
                    vi = plsc.load_gather(vi_r, [rows, dcol])
                    vj = plsc.load_gather(vj_r, [rows, dcol])
                    out = [accs[0] + vi * vj]
                    for k in range(_NEG):
                        ng = plsc.load_gather(ng_r, [ngrows[k], dcol])
                        out.append(accs[k + 1] + vi * ng)
                    accs = tuple(out)
                return accs

            zero = jnp.zeros((16,), jnp.float32)
            accs = lax.fori_loop(0, _DIM // _UNROLL, dstep, (zero,) * 6)
            off = s * _SUB + g * 16
            for t in range(6):
                dots[t, pl.ds(off, 16)] = accs[t]
            return carry2

        lax.fori_loop(0, _SUB // 16, group, 0)

    for t in range(6):
        pltpu.sync_copy(dots.at[t], out_hbm.at[t, pl.ds(wid * _CHUNK, _CHUNK)])


_sc_dots = functools.partial(
    pl.kernel,
    out_type=jax.ShapeDtypeStruct((6, _B), jnp.float32),
    mesh=plsc.VectorSubcoreMesh(core_axis_name="c", subcore_axis_name="s"),
    scratch_types=[
        pltpu.VMEM((_CHUNK,), jnp.int32),
        pltpu.VMEM((_CHUNK,), jnp.int32),
        pltpu.VMEM((_NEG, _CHUNK), jnp.int32),
        pltpu.VMEM((_SUB, _DIM), jnp.float32),
        pltpu.VMEM((_SUB, _DIM), jnp.float32),
        pltpu.VMEM((_SUB * _NEG, _DIM), jnp.float32),
        pltpu.VMEM((_SUB, _DIM), jnp.float32),
        pltpu.VMEM((_SUB, _DIM), jnp.float32),
        pltpu.VMEM((_SUB * _NEG, _DIM), jnp.float32),
        pltpu.VMEM((6, _CHUNK), jnp.float32),
        pltpu.SemaphoreType.DMA,
        pltpu.SemaphoreType.DMA,
    ],
    compiler_params=pltpu.CompilerParams(
        needs_layout_passes=False, use_tc_tiling_on_sc=False),
)(_sc_body)


def _finish_body(dots_ref, out_ref):
    x = dots_ref[...]
    pos = x[0:1, :]
    neg = x[1:6, :]

    def logsig(v):
        return jnp.minimum(v, 0.0) - jnp.log1p(jnp.exp(-jnp.abs(v)))

    tot = jnp.sum(logsig(pos)) + jnp.sum(logsig(-neg))
    out_ref[0, 0] = -tot / _B


_finish = pl.pallas_call(
    _finish_body,
    out_shape=jax.ShapeDtypeStruct((1, 1), jnp.float32),
    out_specs=pl.BlockSpec(memory_space=pltpu.SMEM),
)


def kernel(emb_table, ctx_table, v_i, v_j, negative):
    vi_r = v_i.astype(jnp.int32)
    vj_r = v_j.astype(jnp.int32)
    ng_r = negative.astype(jnp.int32).T
    dots = _sc_dots(vi_r, vj_r, ng_r, emb_table, ctx_table)
    return _finish(dots)[0, 0]
